# baseline (speedup 1.0000x reference)
import functools

import jax
import jax.numpy as jnp
from jax import lax
from jax.experimental import pallas as pl
from jax.experimental.pallas import tpu as pltpu
from jax.experimental.pallas import tpu_sc as plsc

DIM = 64
SEQ = 200
BATCH = 1024
N = BATCH * SEQ
LANES = 16

NC, NS = 2, 16
NW = NC * NS
RPW = N // NW
CHUNK = 128
NCH = RPW // CHUNK
NB = 4

HCHUNK = CHUNK // 2


def _sc_body(x_hbm, tok_hbm, out_hbm, xv, tv, isem, gsem, osem):
    cid = lax.axis_index("c")
    sid = lax.axis_index("s")
    wid = sid * NC + cid
    base_blk = wid * NCH


    def stage_a(g, slot, may_drain):
        blk = base_blk + g
        if may_drain:
            def _drain():
                pltpu.make_async_copy(
                    tv.at[slot],
                    out_hbm.at[pl.ds(blk * CHUNK, CHUNK)],
                    osem.at[slot]).wait()
            pl.when(g >= NB)(_drain)
        pltpu.async_copy(x_hbm.at[blk], xv.at[slot], isem.at[slot])

    def stage_b(g, slot):
        blk = base_blk + g
        pltpu.make_async_copy(x_hbm.at[blk], xv.at[slot], isem.at[slot]).wait()
        pltpu.async_copy(tok_hbm.at[xv.at[slot]], tv.at[slot], gsem.at[slot])

    def stage_c(g, slot):
        blk = base_blk + g
        pltpu.make_async_copy(
            tok_hbm.at[pl.ds(0, CHUNK)], tv.at[slot], gsem.at[slot]).wait()
        pltpu.async_copy(
            tv.at[slot],
            out_hbm.at[pl.ds(blk * CHUNK, CHUNK)],
            osem.at[slot])

    stage_a(0, 0, False)
    stage_a(1, 1, False)
    stage_b(0, 0)

    def outer(g0, carry):
        for b in range(NB):
            g = g0 + b
            stage_c(g, b)
            stage_b(g + 1, (b + 1) % NB)
            stage_a(g + 2, (b + 2) % NB, True)
        return carry

    lax.fori_loop(0, (NCH - 2) // NB, lambda i, c: outer(i * NB, c), 0,
                  unroll=False)

    stage_b(NCH - 1, (NCH - 1) % NB)
    stage_c(NCH - 2, (NCH - 2) % NB)
    stage_c(NCH - 1, (NCH - 1) % NB)
    for g in range(NCH - NB, NCH):
        blk = base_blk + g
        pltpu.make_async_copy(
            tv.at[g % NB],
            out_hbm.at[pl.ds(blk * CHUNK, CHUNK)],
            osem.at[g % NB]).wait()


def _sc_gather(x2, token_table):
    mesh = plsc.VectorSubcoreMesh(
        core_axis_name="c", subcore_axis_name="s", num_cores=NC, num_subcores=NS
    )
    f = pl.kernel(
        _sc_body,
        out_type=jax.ShapeDtypeStruct((N, DIM), jnp.float32),
        mesh=mesh,
        scratch_types=[
            pltpu.VMEM((NB, CHUNK), jnp.int32),
            pltpu.VMEM((NB, CHUNK, DIM), jnp.float32),
            pltpu.SemaphoreType.DMA((NB,)),
            pltpu.SemaphoreType.DMA((NB,)),
            pltpu.SemaphoreType.DMA((NB,)),
        ],
        compiler_params=pltpu.CompilerParams(use_tc_tiling_on_sc=False),
    )
    return f(x2, token_table)


G = 64
PAIRS = SEQ * DIM // 128


def _tc_body(tok_ref, mf_ref, pos2_ref, seg_ref, o_ref):
    tok = tok_ref[...]
    pos_t = jnp.broadcast_to(
        pos2_ref[...][None], (G, PAIRS, 2 * DIM)).reshape(G * PAIRS, 2 * DIM)
    ml = mf_ref[:, 0:1].astype(jnp.float32)
    mr = mf_ref[:, 1:2].astype(jnp.float32)
    lane = lax.broadcasted_iota(jnp.int32, (G * PAIRS, 2 * DIM), 1)
    m_eff = jnp.where(lane < DIM, ml, mr)
    s0 = seg_ref[0:1, :]
    d = seg_ref[1:2, :] - s0
    s0c = jnp.concatenate([s0, s0], axis=1)
    dc = jnp.concatenate([d, d], axis=1)
    res = tok + pos_t + s0c + m_eff * dc
    left = res[:, 0:DIM]
    right = res[:, DIM:2 * DIM]
    inter = jnp.stack([left, right], axis=1)
    o_ref[...] = inter.reshape(G, SEQ, DIM)


def _tc_finish(tok2, mf, pos2, seg_table):
    return pl.pallas_call(
        _tc_body,
        grid=(BATCH // G,),
        in_specs=[
            pl.BlockSpec((G * PAIRS, 2 * DIM), lambda i: (i, 0)),
            pl.BlockSpec((G * PAIRS, 2), lambda i: (i, 0)),
            pl.BlockSpec((PAIRS, 2 * DIM), lambda i: (0, 0)),
            pl.BlockSpec((2, DIM), lambda i: (0, 0)),
        ],
        out_specs=pl.BlockSpec((G, SEQ, DIM), lambda i: (i, 0, 0)),
        out_shape=jax.ShapeDtypeStruct((BATCH, SEQ, DIM), jnp.float32),
    )(tok2, mf, pos2, seg_table)


def kernel(x, segment_mask, token_table, pos_table, seg_table):
    x = x.astype(jnp.int32)
    segment_mask = segment_mask.astype(jnp.int32)
    x2 = x.reshape(N // CHUNK, CHUNK)
    tok2 = _sc_gather(x2, token_table).reshape(N // 2, 2 * DIM)
    mf = segment_mask.reshape(N // 2, 2)
    pos2 = pos_table[:SEQ].reshape(PAIRS, 2 * DIM)
    return _tc_finish(tok2, mf, pos2, seg_table)


# speedup vs baseline: 1.2886x; 1.2886x over previous
import functools

import jax
import jax.numpy as jnp
from jax import lax
from jax.experimental import pallas as pl
from jax.experimental.pallas import tpu as pltpu
from jax.experimental.pallas import tpu_sc as plsc

DIM = 64
SEQ = 200
BATCH = 1024
N = BATCH * SEQ
LANES = 16

NC, NS = 2, 16
NW = NC * NS
RPW = N // NW
CHUNK = 128
NCH = RPW // CHUNK
NB = 4


def _sc_body(x_hbm, tok_hbm, out_hbm, xv, tv, isem, gsem, osem):
    cid = lax.axis_index("c")
    sid = lax.axis_index("s")
    wid = sid * NC + cid
    base_blk = wid * NCH

    def out_dst(blk):
        return out_hbm.at[pl.ds(blk * CHUNK, CHUNK), pl.ds(0, DIM)]


    def stage_a(g, slot, may_drain):
        blk = base_blk + g
        if may_drain:
            def _drain():
                pltpu.make_async_copy(tv.at[slot], out_dst(blk),
                                      osem.at[slot]).wait()
            pl.when(g >= NB)(_drain)
        pltpu.async_copy(x_hbm.at[blk], xv.at[slot], isem.at[slot])

    def stage_b(g, slot):
        blk = base_blk + g
        pltpu.make_async_copy(x_hbm.at[blk], xv.at[slot], isem.at[slot]).wait()
        pltpu.async_copy(tok_hbm.at[xv.at[slot]], tv.at[slot], gsem.at[slot])

    def stage_c(g, slot):
        blk = base_blk + g
        pltpu.make_async_copy(
            tok_hbm.at[pl.ds(0, CHUNK)], tv.at[slot], gsem.at[slot]).wait()
        pltpu.async_copy(tv.at[slot], out_dst(blk), osem.at[slot])

    stage_a(0, 0, False)
    stage_a(1, 1, False)
    stage_b(0, 0)

    def outer(g0, carry):
        for b in range(NB):
            g = g0 + b
            stage_c(g, b)
            stage_b(g + 1, (b + 1) % NB)
            stage_a(g + 2, (b + 2) % NB, True)
        return carry

    lax.fori_loop(0, (NCH - 2) // NB, lambda i, c: outer(i * NB, c), 0,
                  unroll=False)

    stage_b(NCH - 1, (NCH - 1) % NB)
    stage_c(NCH - 2, (NCH - 2) % NB)
    stage_c(NCH - 1, (NCH - 1) % NB)
    for g in range(NCH - NB, NCH):
        blk = base_blk + g
        pltpu.make_async_copy(tv.at[g % NB], out_dst(blk),
                              osem.at[g % NB]).wait()


def _sc_gather(x2, token_table):
    mesh = plsc.VectorSubcoreMesh(
        core_axis_name="c", subcore_axis_name="s", num_cores=NC, num_subcores=NS
    )
    f = pl.kernel(
        _sc_body,
        out_type=jax.ShapeDtypeStruct((N, 2 * DIM), jnp.float32),
        mesh=mesh,
        scratch_types=[
            pltpu.VMEM((NB, CHUNK), jnp.int32),
            pltpu.VMEM((NB, CHUNK, DIM), jnp.float32),
            pltpu.SemaphoreType.DMA((NB,)),
            pltpu.SemaphoreType.DMA((NB,)),
            pltpu.SemaphoreType.DMA((NB,)),
        ],
        compiler_params=pltpu.CompilerParams(use_tc_tiling_on_sc=False),
    )
    return f(x2, token_table)


G = 64


def _tc_body(tok_ref, mask_ref, pos_ref, seg_ref, o_ref):
    tok = tok_ref[:, :, 0:DIM]
    m = mask_ref[...].astype(jnp.float32)[..., None]
    pos = pos_ref[0:SEQ, :][None]
    s0 = seg_ref[0:1, :][None]
    d = seg_ref[1:2, :][None] - s0
    o_ref[...] = tok + pos + s0 + m * d


def _tc_finish(tok3, segment_mask, pos_table, seg_table):
    return pl.pallas_call(
        _tc_body,
        grid=(BATCH // G,),
        in_specs=[
            pl.BlockSpec((G, SEQ, 2 * DIM), lambda i: (i, 0, 0)),
            pl.BlockSpec((G, SEQ), lambda i: (i, 0)),
            pl.BlockSpec((512, DIM), lambda i: (0, 0)),
            pl.BlockSpec((2, DIM), lambda i: (0, 0)),
        ],
        out_specs=pl.BlockSpec((G, SEQ, DIM), lambda i: (i, 0, 0)),
        out_shape=jax.ShapeDtypeStruct((BATCH, SEQ, DIM), jnp.float32),
    )(tok3, segment_mask, pos_table, seg_table)


def kernel(x, segment_mask, token_table, pos_table, seg_table):
    x = x.astype(jnp.int32)
    segment_mask = segment_mask.astype(jnp.int32)
    x2 = x.reshape(N // CHUNK, CHUNK)
    tok = _sc_gather(x2, token_table)
    tok3 = tok.reshape(BATCH, SEQ, 2 * DIM)
    return _tc_finish(tok3, segment_mask, pos_table, seg_table)


# speedup vs baseline: 1.7693x; 1.3731x over previous
import functools

import jax
import jax.numpy as jnp
from jax import lax
from jax.experimental import pallas as pl
from jax.experimental.pallas import tpu as pltpu
from jax.experimental.pallas import tpu_sc as plsc

DIM = 64
SEQ = 200
BATCH = 1024
N = BATCH * SEQ
LANES = 16

NC, NS = 2, 16
NW = NC * NS
RPW = N // NW
CHUNK = 128
NCH = RPW // CHUNK
NB = 4


def _sc_body(x_hbm, tok_hbm, out_hbm, xv, tv, isem, gsem, osem):
    cid = lax.axis_index("c")
    sid = lax.axis_index("s")
    wid = sid * NC + cid
    base_blk = wid * NCH

    def out_dst(blk):
        return out_hbm.at[pl.ds(blk * CHUNK, CHUNK), pl.ds(0, DIM)]


    def stage_a(g, slot, may_drain):
        blk = base_blk + g
        if may_drain:
            def _drain():
                pltpu.make_async_copy(tv.at[slot], out_dst(blk),
                                      osem.at[slot]).wait()
            pl.when(g >= NB)(_drain)
        pltpu.async_copy(x_hbm.at[blk], xv.at[slot], isem.at[slot])

    def stage_b(g, slot):
        blk = base_blk + g
        pltpu.make_async_copy(x_hbm.at[blk], xv.at[slot], isem.at[slot]).wait()
        pltpu.async_copy(tok_hbm.at[xv.at[slot]], tv.at[slot], gsem.at[slot])

    def stage_c(g, slot):
        blk = base_blk + g
        pltpu.make_async_copy(
            tok_hbm.at[pl.ds(0, CHUNK)], tv.at[slot], gsem.at[slot]).wait()
        pltpu.async_copy(tv.at[slot], out_dst(blk), osem.at[slot])

    stage_a(0, 0, False)
    stage_a(1, 1, False)
    stage_b(0, 0)

    def outer(g0, carry):
        for b in range(NB):
            g = g0 + b
            stage_c(g, b)
            stage_b(g + 1, (b + 1) % NB)
            stage_a(g + 2, (b + 2) % NB, True)
        return carry

    lax.fori_loop(0, (NCH - 2) // NB, lambda i, c: outer(i * NB, c), 0,
                  unroll=False)

    stage_b(NCH - 1, (NCH - 1) % NB)
    stage_c(NCH - 2, (NCH - 2) % NB)
    stage_c(NCH - 1, (NCH - 1) % NB)
    for g in range(NCH - NB, NCH):
        blk = base_blk + g
        pltpu.make_async_copy(tv.at[g % NB], out_dst(blk),
                              osem.at[g % NB]).wait()


def _sc_gather(x2, token_table):
    mesh = plsc.VectorSubcoreMesh(
        core_axis_name="c", subcore_axis_name="s", num_cores=NC, num_subcores=NS
    )
    f = pl.kernel(
        _sc_body,
        out_type=jax.ShapeDtypeStruct((N, 2 * DIM), jnp.float32),
        mesh=mesh,
        scratch_types=[
            pltpu.VMEM((NB, CHUNK), jnp.int32),
            pltpu.VMEM((NB, CHUNK, DIM), jnp.float32),
            pltpu.SemaphoreType.DMA((NB,)),
            pltpu.SemaphoreType.DMA((NB,)),
            pltpu.SemaphoreType.DMA((NB,)),
        ],
        compiler_params=pltpu.CompilerParams(use_tc_tiling_on_sc=False),
    )
    return f(x2, token_table)


SB = 8


def _tc_body(tok_ref, maskt_ref, pos_ref, seg_ref, o_ref):
    tok = tok_ref[:, :, 0:DIM]
    tt = jnp.transpose(tok, (0, 2, 1))
    m = maskt_ref[...].astype(jnp.float32)[:, None, :]
    p3 = pos_ref[...][:, :, None]
    s0 = seg_ref[0:1, :]
    d = seg_ref[1:2, :] - s0
    s03 = s0[:, :, None]
    d3 = d[:, :, None]
    o_ref[...] = tt + p3 + s03 + m * d3


def _tc_finish(tok3, mask_t, pos_t, seg_table):
    return pl.pallas_call(
        _tc_body,
        grid=(SEQ // SB,),
        in_specs=[
            pl.BlockSpec((SB, BATCH, 2 * DIM), lambda i: (i, 0, 0)),
            pl.BlockSpec((SB, BATCH), lambda i: (i, 0)),
            pl.BlockSpec((SB, DIM), lambda i: (i, 0)),
            pl.BlockSpec((2, DIM), lambda i: (0, 0)),
        ],
        out_specs=pl.BlockSpec((SB, DIM, BATCH), lambda i: (i, 0, 0)),
        out_shape=jax.ShapeDtypeStruct((SEQ, DIM, BATCH), jnp.float32),
    )(tok3, mask_t, pos_t, seg_table)


def kernel(x, segment_mask, token_table, pos_table, seg_table):
    x = x.astype(jnp.int32)
    segment_mask = segment_mask.astype(jnp.int32)
    xs = x.T.reshape(N // CHUNK, CHUNK)
    tok = _sc_gather(xs, token_table)
    tok3 = tok.reshape(SEQ, BATCH, 2 * DIM)
    mask_t = segment_mask.T
    out_t = _tc_finish(tok3, mask_t, pos_table[:SEQ], seg_table)
    return out_t.transpose(2, 0, 1)


# speedup vs baseline: 1.8472x; 1.0440x over previous
import functools

import jax
import jax.numpy as jnp
from jax import lax
from jax.experimental import pallas as pl
from jax.experimental.pallas import tpu as pltpu
from jax.experimental.pallas import tpu_sc as plsc

DIM = 64
SEQ = 200
BATCH = 1024
N = BATCH * SEQ
LANES = 16

NC, NS = 2, 16
NW = NC * NS
CHUNK = 128
NB = 4

SPLIT = 96


def _make_sc_body(h_base, nch):
    def _sc_body(x_hbm, tok_hbm, out_hbm, xv, tv, isem, gsem, osem):
        cid = lax.axis_index("c")
        sid = lax.axis_index("s")
        wid = sid * NC + cid
        base_blk = wid * nch

        def out_dst(blk):
            return out_hbm.at[pl.ds(blk * CHUNK, CHUNK), pl.ds(0, DIM)]


        def stage_a(g, slot, may_drain):
            blk = base_blk + g
            if may_drain:
                def _drain():
                    pltpu.make_async_copy(tv.at[slot], out_dst(blk),
                                          osem.at[slot]).wait()
                pl.when(g >= NB)(_drain)
            pltpu.async_copy(x_hbm.at[h_base + blk], xv.at[slot],
                             isem.at[slot])

        def stage_b(g, slot):
            blk = base_blk + g
            pltpu.make_async_copy(x_hbm.at[h_base + blk], xv.at[slot],
                                  isem.at[slot]).wait()
            pltpu.async_copy(tok_hbm.at[xv.at[slot]], tv.at[slot],
                             gsem.at[slot])

        def stage_c(g, slot):
            blk = base_blk + g
            pltpu.make_async_copy(
                tok_hbm.at[pl.ds(0, CHUNK)], tv.at[slot], gsem.at[slot]).wait()
            pltpu.async_copy(tv.at[slot], out_dst(blk), osem.at[slot])

        stage_a(0, 0, False)
        stage_a(1, 1, False)
        stage_b(0, 0)

        steady = ((nch - 2) // NB) * NB

        def outer(g0, carry):
            for b in range(NB):
                g = g0 + b
                stage_c(g, b)
                stage_b(g + 1, (b + 1) % NB)
                stage_a(g + 2, (b + 2) % NB, True)
            return carry

        lax.fori_loop(0, steady // NB, lambda i, c: outer(i * NB, c), 0,
                      unroll=False)

        for g in range(steady, nch):
            stage_c(g, g % NB)
            if g + 1 < nch:
                stage_b(g + 1, (g + 1) % NB)
            if g + 2 < nch:
                stage_a(g + 2, (g + 2) % NB, g + 2 >= NB)
        for g in range(nch - NB, nch):
            blk = base_blk + g
            pltpu.make_async_copy(tv.at[g % NB], out_dst(blk),
                                  osem.at[g % NB]).wait()

    return _sc_body


def _sc_gather(xs, token_table, h_base, n_tok):
    nch = n_tok // (NW * CHUNK)
    mesh = plsc.VectorSubcoreMesh(
        core_axis_name="c", subcore_axis_name="s", num_cores=NC, num_subcores=NS
    )
    f = pl.kernel(
        _make_sc_body(h_base, nch),
        out_type=jax.ShapeDtypeStruct((n_tok, 2 * DIM), jnp.float32),
        mesh=mesh,
        scratch_types=[
            pltpu.VMEM((NB, CHUNK), jnp.int32),
            pltpu.VMEM((NB, CHUNK, DIM), jnp.float32),
            pltpu.SemaphoreType.DMA((NB,)),
            pltpu.SemaphoreType.DMA((NB,)),
            pltpu.SemaphoreType.DMA((NB,)),
        ],
        compiler_params=pltpu.CompilerParams(use_tc_tiling_on_sc=False),
    )
    return f(xs, token_table)


SB = 8


def _tc_body(tok_ref, maskt_ref, pos_ref, seg_ref, o_ref):
    tok = tok_ref[:, :, 0:DIM]
    tt = jnp.transpose(tok, (0, 2, 1))
    m = maskt_ref[...].astype(jnp.float32)[:, None, :]
    p3 = pos_ref[...][:, :, None]
    s0 = seg_ref[0:1, :]
    d = seg_ref[1:2, :] - s0
    s03 = s0[:, :, None]
    d3 = d[:, :, None]
    o_ref[...] = tt + p3 + s03 + m * d3


def _tc_body_aliased(tok_ref, maskt_ref, pos_ref, seg_ref, prev_ref, o_ref):
    _tc_body(tok_ref, maskt_ref, pos_ref, seg_ref, o_ref)


def _tc_finish_first(tok3, mask_t, pos, seg_table):
    nsb = tok3.shape[0] // SB
    return pl.pallas_call(
        _tc_body,
        grid=(nsb,),
        in_specs=[
            pl.BlockSpec((SB, BATCH, 2 * DIM), lambda i: (i, 0, 0)),
            pl.BlockSpec((SB, BATCH), lambda i: (i, 0)),
            pl.BlockSpec((SB, DIM), lambda i: (i, 0)),
            pl.BlockSpec((2, DIM), lambda i: (0, 0)),
        ],
        out_specs=pl.BlockSpec((SB, DIM, BATCH), lambda i: (i, 0, 0)),
        out_shape=jax.ShapeDtypeStruct((SEQ, DIM, BATCH), jnp.float32),
    )(tok3, mask_t, pos, seg_table)


def _tc_finish_second(tok3, mask_t, pos, seg_table, prev):
    nsb = tok3.shape[0] // SB
    off = SPLIT // SB
    return pl.pallas_call(
        _tc_body_aliased,
        grid=(nsb,),
        in_specs=[
            pl.BlockSpec((SB, BATCH, 2 * DIM), lambda i: (i, 0, 0)),
            pl.BlockSpec((SB, BATCH), lambda i: (i, 0)),
            pl.BlockSpec((SB, DIM), lambda i: (i, 0)),
            pl.BlockSpec((2, DIM), lambda i: (0, 0)),
            pl.BlockSpec(memory_space=pl.ANY),
        ],
        out_specs=pl.BlockSpec((SB, DIM, BATCH), lambda i: (i + off, 0, 0)),
        out_shape=jax.ShapeDtypeStruct((SEQ, DIM, BATCH), jnp.float32),
        input_output_aliases={4: 0},
    )(tok3, mask_t, pos, seg_table, prev)


def kernel(x, segment_mask, token_table, pos_table, seg_table):
    x = x.astype(jnp.int32)
    segment_mask = segment_mask.astype(jnp.int32)
    xs = x.T.reshape(N // CHUNK, CHUNK)
    mask_t = segment_mask.T
    pos = pos_table[:SEQ]

    na, nb_ = SPLIT * BATCH, (SEQ - SPLIT) * BATCH
    tok_a = _sc_gather(xs, token_table, 0, na)
    tok_b = _sc_gather(xs, token_table, na // CHUNK, nb_)
    tok3_a = tok_a.reshape(SPLIT, BATCH, 2 * DIM)
    tok3_b = tok_b.reshape(SEQ - SPLIT, BATCH, 2 * DIM)

    out_a = _tc_finish_first(tok3_a, mask_t[:SPLIT], pos[:SPLIT], seg_table)
    out_t = _tc_finish_second(tok3_b, mask_t[SPLIT:], pos[SPLIT:], seg_table,
                              out_a)
    return out_t.transpose(2, 0, 1)


# speedup vs baseline: 1.9270x; 1.0432x over previous
import functools

import jax
import jax.numpy as jnp
from jax import lax
from jax.experimental import pallas as pl
from jax.experimental.pallas import tpu as pltpu
from jax.experimental.pallas import tpu_sc as plsc

DIM = 64
SEQ = 200
BATCH = 1024
N = BATCH * SEQ
LANES = 16

NC, NS = 2, 16
NW = NC * NS
CHUNK = 128
NB = 4

SPLIT = 96


def _make_sc_body(h_base, nch):
    def _sc_body(x_hbm, tok_hbm, out_hbm, xv, tv, isem, gsem, osem):
        cid = lax.axis_index("c")
        sid = lax.axis_index("s")
        wid = sid * NC + cid
        base_blk = wid * nch

        def out_dst(blk):
            return out_hbm.at[pl.ds(blk * CHUNK, CHUNK), pl.ds(0, DIM)]


        def stage_a(g, slot, may_drain):
            blk = base_blk + g
            if may_drain:
                def _drain():
                    pltpu.make_async_copy(tv.at[slot], out_dst(blk),
                                          osem.at[slot]).wait()
                pl.when(g >= NB)(_drain)
            pltpu.async_copy(x_hbm.at[h_base + blk], xv.at[slot],
                             isem.at[slot])

        def stage_b(g, slot):
            blk = base_blk + g
            pltpu.make_async_copy(x_hbm.at[h_base + blk], xv.at[slot],
                                  isem.at[slot]).wait()
            for j in range(CHUNK // LANES):
                sl = pl.ds(j * LANES, LANES)
                xv[slot, sl] = xv[slot, sl] * 2
            pltpu.async_copy(tok_hbm.at[xv.at[slot]], tv.at[slot],
                             gsem.at[slot])

        def stage_c(g, slot):
            blk = base_blk + g
            pltpu.make_async_copy(
                tok_hbm.at[pl.ds(0, CHUNK)], tv.at[slot], gsem.at[slot]).wait()
            pltpu.async_copy(tv.at[slot], out_dst(blk), osem.at[slot])

        stage_a(0, 0, False)
        stage_a(1, 1, False)
        stage_b(0, 0)

        steady = ((nch - 2) // NB) * NB

        def outer(g0, carry):
            for b in range(NB):
                g = g0 + b
                stage_c(g, b)
                stage_b(g + 1, (b + 1) % NB)
                stage_a(g + 2, (b + 2) % NB, True)
            return carry

        lax.fori_loop(0, steady // NB, lambda i, c: outer(i * NB, c), 0,
                      unroll=False)

        for g in range(steady, nch):
            stage_c(g, g % NB)
            if g + 1 < nch:
                stage_b(g + 1, (g + 1) % NB)
            if g + 2 < nch:
                stage_a(g + 2, (g + 2) % NB, g + 2 >= NB)
        for g in range(nch - NB, nch):
            blk = base_blk + g
            pltpu.make_async_copy(tv.at[g % NB], out_dst(blk),
                                  osem.at[g % NB]).wait()

    return _sc_body


def _sc_gather(xs, table_lin, h_base, n_tok):
    nch = n_tok // (NW * CHUNK)
    mesh = plsc.VectorSubcoreMesh(
        core_axis_name="c", subcore_axis_name="s", num_cores=NC, num_subcores=NS
    )
    f = pl.kernel(
        _make_sc_body(h_base, nch),
        out_type=jax.ShapeDtypeStruct((n_tok, 2 * DIM), jnp.float32),
        mesh=mesh,
        scratch_types=[
            pltpu.VMEM((NB, CHUNK), jnp.int32),
            pltpu.VMEM((NB, CHUNK, DIM), jnp.float32),
            pltpu.SemaphoreType.DMA((NB,)),
            pltpu.SemaphoreType.DMA((NB,)),
            pltpu.SemaphoreType.DMA((NB,)),
        ],
        compiler_params=pltpu.CompilerParams(use_tc_tiling_on_sc=False),
    )
    return f(xs, table_lin)


VOCAB = 100000
VB = 2048


def _tt_body(tt_ref, o_ref):
    tr = jnp.transpose(tt_ref[...], (1, 0))
    o_ref[:, 0:DIM] = tr
    o_ref[:, DIM:2 * DIM] = tr


def _table_transpose(table_t):
    nvb = (VOCAB + VB - 1) // VB
    return pl.pallas_call(
        _tt_body,
        grid=(nvb,),
        in_specs=[pl.BlockSpec((DIM, VB), lambda i: (0, i))],
        out_specs=pl.BlockSpec((VB, 2 * DIM), lambda i: (i, 0)),
        out_shape=jax.ShapeDtypeStruct((VOCAB, 2 * DIM), jnp.float32),
    )(table_t)


SB = 8


def _tc_body(tok_ref, maskt_ref, pos_ref, seg_ref, o_ref):
    tok = tok_ref[:, :, 0:DIM]
    tt = jnp.transpose(tok, (0, 2, 1))
    m = maskt_ref[...].astype(jnp.float32)[:, None, :]
    p3 = pos_ref[...][:, :, None]
    s0 = seg_ref[0:1, :]
    d = seg_ref[1:2, :] - s0
    s03 = s0[:, :, None]
    d3 = d[:, :, None]
    o_ref[...] = tt + p3 + s03 + m * d3


def _tc_body_aliased(tok_ref, maskt_ref, pos_ref, seg_ref, prev_ref, o_ref):
    _tc_body(tok_ref, maskt_ref, pos_ref, seg_ref, o_ref)


def _tc_finish_first(tok3, mask_t, pos, seg_table):
    nsb = tok3.shape[0] // SB
    return pl.pallas_call(
        _tc_body,
        grid=(nsb,),
        in_specs=[
            pl.BlockSpec((SB, BATCH, 2 * DIM), lambda i: (i, 0, 0)),
            pl.BlockSpec((SB, BATCH), lambda i: (i, 0)),
            pl.BlockSpec((SB, DIM), lambda i: (i, 0)),
            pl.BlockSpec((2, DIM), lambda i: (0, 0)),
        ],
        out_specs=pl.BlockSpec((SB, DIM, BATCH), lambda i: (i, 0, 0)),
        out_shape=jax.ShapeDtypeStruct((SEQ, DIM, BATCH), jnp.float32),
    )(tok3, mask_t, pos, seg_table)


def _tc_finish_second(tok3, mask_t, pos, seg_table, prev):
    nsb = tok3.shape[0] // SB
    off = SPLIT // SB
    return pl.pallas_call(
        _tc_body_aliased,
        grid=(nsb,),
        in_specs=[
            pl.BlockSpec((SB, BATCH, 2 * DIM), lambda i: (i, 0, 0)),
            pl.BlockSpec((SB, BATCH), lambda i: (i, 0)),
            pl.BlockSpec((SB, DIM), lambda i: (i, 0)),
            pl.BlockSpec((2, DIM), lambda i: (0, 0)),
            pl.BlockSpec(memory_space=pl.ANY),
        ],
        out_specs=pl.BlockSpec((SB, DIM, BATCH), lambda i: (i + off, 0, 0)),
        out_shape=jax.ShapeDtypeStruct((SEQ, DIM, BATCH), jnp.float32),
        input_output_aliases={4: 0},
    )(tok3, mask_t, pos, seg_table, prev)


def kernel(x, segment_mask, token_table, pos_table, seg_table):
    x = x.astype(jnp.int32)
    segment_mask = segment_mask.astype(jnp.int32)
    xs = x.T.reshape(N // CHUNK, CHUNK)
    mask_t = segment_mask.T
    pos = pos_table[:SEQ]

    table_lin = _table_transpose(token_table.T).reshape(2 * VOCAB, DIM)

    na, nb_ = SPLIT * BATCH, (SEQ - SPLIT) * BATCH
    tok_a = _sc_gather(xs, table_lin, 0, na)
    tok_b = _sc_gather(xs, table_lin, na // CHUNK, nb_)
    tok3_a = tok_a.reshape(SPLIT, BATCH, 2 * DIM)
    tok3_b = tok_b.reshape(SEQ - SPLIT, BATCH, 2 * DIM)

    out_a = _tc_finish_first(tok3_a, mask_t[:SPLIT], pos[:SPLIT], seg_table)
    out_t = _tc_finish_second(tok3_b, mask_t[SPLIT:], pos[SPLIT:], seg_table,
                              out_a)
    return out_t.transpose(2, 0, 1)


# speedup vs baseline: 1.9301x; 1.0016x over previous
import functools

import jax
import jax.numpy as jnp
from jax import lax
from jax.experimental import pallas as pl
from jax.experimental.pallas import tpu as pltpu
from jax.experimental.pallas import tpu_sc as plsc

DIM = 64
SEQ = 200
BATCH = 1024
N = BATCH * SEQ
LANES = 16

NC, NS = 2, 16
NW = NC * NS
CHUNK = 128
NB = 4

SPLIT = 96


def _make_sc_body(h_base, nch):
    def _sc_body(x_hbm, tok_hbm, out_hbm, xv, tv, isem, gsem, osem):
        cid = lax.axis_index("c")
        sid = lax.axis_index("s")
        wid = sid * NC + cid
        base_blk = wid * nch

        def out_dst(blk):
            return out_hbm.at[pl.ds(blk * CHUNK, CHUNK), pl.ds(0, DIM)]


        def stage_a(g, slot, may_drain):
            blk = base_blk + g
            if may_drain:
                def _drain():
                    pltpu.make_async_copy(tv.at[slot], out_dst(blk),
                                          osem.at[slot]).wait()
                pl.when(g >= NB)(_drain)
            pltpu.async_copy(x_hbm.at[h_base + blk], xv.at[slot],
                             isem.at[slot])

        def stage_b(g, slot):
            blk = base_blk + g
            pltpu.make_async_copy(x_hbm.at[h_base + blk], xv.at[slot],
                                  isem.at[slot]).wait()
            for j in range(CHUNK // LANES):
                sl = pl.ds(j * LANES, LANES)
                xv[slot, sl] = xv[slot, sl] * 2
            pltpu.async_copy(tok_hbm.at[xv.at[slot]], tv.at[slot],
                             gsem.at[slot])

        def stage_c(g, slot):
            blk = base_blk + g
            pltpu.make_async_copy(
                tok_hbm.at[pl.ds(0, CHUNK)], tv.at[slot], gsem.at[slot]).wait()
            pltpu.async_copy(tv.at[slot], out_dst(blk), osem.at[slot])

        stage_a(0, 0, False)
        stage_a(1, 1, False)
        stage_b(0, 0)

        steady = ((nch - 2) // NB) * NB

        def outer(g0, carry):
            for b in range(NB):
                g = g0 + b
                stage_c(g, b)
                stage_b(g + 1, (b + 1) % NB)
                stage_a(g + 2, (b + 2) % NB, True)
            return carry

        lax.fori_loop(0, steady // NB, lambda i, c: outer(i * NB, c), 0,
                      unroll=False)

        for g in range(steady, nch):
            stage_c(g, g % NB)
            if g + 1 < nch:
                stage_b(g + 1, (g + 1) % NB)
            if g + 2 < nch:
                stage_a(g + 2, (g + 2) % NB, g + 2 >= NB)
        for g in range(nch - NB, nch):
            blk = base_blk + g
            pltpu.make_async_copy(tv.at[g % NB], out_dst(blk),
                                  osem.at[g % NB]).wait()

    return _sc_body


def _sc_gather(xs, table_lin, h_base, n_tok):
    nch = n_tok // (NW * CHUNK)
    mesh = plsc.VectorSubcoreMesh(
        core_axis_name="c", subcore_axis_name="s", num_cores=NC, num_subcores=NS
    )
    f = pl.kernel(
        _make_sc_body(h_base, nch),
        out_type=jax.ShapeDtypeStruct((n_tok, 2 * DIM), jnp.float32),
        mesh=mesh,
        scratch_types=[
            pltpu.VMEM((NB, CHUNK), jnp.int32),
            pltpu.VMEM((NB, CHUNK, DIM), jnp.float32),
            pltpu.SemaphoreType.DMA((NB,)),
            pltpu.SemaphoreType.DMA((NB,)),
            pltpu.SemaphoreType.DMA((NB,)),
        ],
        compiler_params=pltpu.CompilerParams(use_tc_tiling_on_sc=False),
    )
    return f(xs, table_lin)


VOCAB = 100000
VB = 2048


def _tt_body(tt_ref, o_ref):
    tr = jnp.transpose(tt_ref[...], (1, 0))
    o_ref[:, 0:DIM] = tr
    o_ref[:, DIM:2 * DIM] = tr


def _table_transpose(table_t):
    nvb = (VOCAB + VB - 1) // VB
    return pl.pallas_call(
        _tt_body,
        grid=(nvb,),
        in_specs=[pl.BlockSpec((DIM, VB), lambda i: (0, i))],
        out_specs=pl.BlockSpec((VB, 2 * DIM), lambda i: (i, 0)),
        out_shape=jax.ShapeDtypeStruct((VOCAB, 2 * DIM), jnp.float32),
    )(table_t)


SB = 8


def _tc_body(tok_ref, maskt_ref, pos_ref, seg_ref, o_ref):
    tok = tok_ref[:, :, 0:DIM]
    tt = jnp.transpose(tok, (0, 2, 1))
    m = maskt_ref[...].astype(jnp.float32)[:, None, :]
    p3 = pos_ref[...][:, :, None]
    s0 = seg_ref[0:1, :]
    d = seg_ref[1:2, :] - s0
    s03 = s0[:, :, None]
    d3 = d[:, :, None]
    o_ref[...] = tt + p3 + s03 + m * d3


def _tc_body_aliased(tok_ref, maskt_ref, pos_ref, seg_ref, prev_ref, o_ref):
    _tc_body(tok_ref, maskt_ref, pos_ref, seg_ref, o_ref)


def _tc_finish(tok3, mask_t, pos, seg_table, s_off, prev=None):
    nsb = tok3.shape[0] // SB
    off = s_off // SB
    specs = [
        pl.BlockSpec((SB, BATCH, 2 * DIM), lambda i: (i, 0, 0)),
        pl.BlockSpec((SB, BATCH), lambda i: (i, 0)),
        pl.BlockSpec((SB, DIM), lambda i: (i, 0)),
        pl.BlockSpec((2, DIM), lambda i: (0, 0)),
    ]
    args = [tok3, mask_t, pos, seg_table]
    body = _tc_body
    aliases = {}
    if prev is not None:
        specs.append(pl.BlockSpec(memory_space=pl.ANY))
        args.append(prev)
        body = _tc_body_aliased
        aliases = {4: 0}
    return pl.pallas_call(
        body,
        grid=(nsb,),
        in_specs=specs,
        out_specs=pl.BlockSpec((SB, DIM, BATCH), lambda i: (i + off, 0, 0)),
        out_shape=jax.ShapeDtypeStruct((SEQ, DIM, BATCH), jnp.float32),
        input_output_aliases=aliases,
    )(*args)


SPLITS = (48, 48, 48, 56)


def kernel(x, segment_mask, token_table, pos_table, seg_table):
    x = x.astype(jnp.int32)
    segment_mask = segment_mask.astype(jnp.int32)
    xs = x.T.reshape(N // CHUNK, CHUNK)
    mask_t = segment_mask.T
    pos = pos_table[:SEQ]

    table_lin = _table_transpose(token_table.T).reshape(2 * VOCAB, DIM)

    toks = []
    s0 = 0
    for ns in SPLITS:
        n_tok = ns * BATCH
        tok = _sc_gather(xs, table_lin, s0 * BATCH // CHUNK, n_tok)
        toks.append(tok.reshape(ns, BATCH, 2 * DIM))
        s0 += ns

    out = None
    s0 = 0
    for ns, tok3 in zip(SPLITS, toks):
        out = _tc_finish(tok3, mask_t[s0:s0 + ns], pos[s0:s0 + ns], seg_table,
                         s0, out)
        s0 += ns
    return out.transpose(2, 0, 1)


# speedup vs baseline: 2.0054x; 1.0390x over previous
import functools

import jax
import jax.numpy as jnp
from jax import lax
from jax.experimental import pallas as pl
from jax.experimental.pallas import tpu as pltpu
from jax.experimental.pallas import tpu_sc as plsc

DIM = 64
SEQ = 200
BATCH = 1024
N = BATCH * SEQ
LANES = 16

NC, NS = 2, 16
NW = NC * NS
CHUNK = 128
NB = 4

SPLIT = 96


def _make_sc_body(h_base, nch):
    def _sc_body(x_hbm, tok_hbm, out_hbm, xv, tv, isem, gsem, osem):
        cid = lax.axis_index("c")
        sid = lax.axis_index("s")
        wid = sid * NC + cid
        base_blk = wid * nch

        def out_dst(blk):
            return out_hbm.at[pl.ds(blk * CHUNK, CHUNK), pl.ds(0, DIM)]


        def stage_a(g, slot, may_drain):
            blk = base_blk + g
            if may_drain:
                def _drain():
                    pltpu.make_async_copy(tv.at[slot], out_dst(blk),
                                          osem.at[slot]).wait()
                pl.when(g >= NB)(_drain)
            pltpu.async_copy(x_hbm.at[h_base + blk], xv.at[slot],
                             isem.at[slot])

        def stage_b(g, slot):
            blk = base_blk + g
            pltpu.make_async_copy(x_hbm.at[h_base + blk], xv.at[slot],
                                  isem.at[slot]).wait()
            for j in range(CHUNK // LANES):
                sl = pl.ds(j * LANES, LANES)
                v = xv[slot, sl]
                xv[slot, sl] = ((v & jnp.int32(~2047))
                                + ((v & jnp.int32(1023)) << 1)
                                + ((v >> 10) & jnp.int32(1)))
            pltpu.async_copy(tok_hbm.at[xv.at[slot]], tv.at[slot],
                             gsem.at[slot])

        def stage_c(g, slot):
            blk = base_blk + g
            pltpu.make_async_copy(
                tok_hbm.at[pl.ds(0, CHUNK)], tv.at[slot], gsem.at[slot]).wait()
            pltpu.async_copy(tv.at[slot], out_dst(blk), osem.at[slot])

        stage_a(0, 0, False)
        stage_a(1, 1, False)
        stage_b(0, 0)

        steady = ((nch - 2) // NB) * NB

        def outer(g0, carry):
            for b in range(NB):
                g = g0 + b
                stage_c(g, b)
                stage_b(g + 1, (b + 1) % NB)
                stage_a(g + 2, (b + 2) % NB, True)
            return carry

        lax.fori_loop(0, steady // NB, lambda i, c: outer(i * NB, c), 0,
                      unroll=False)

        for g in range(steady, nch):
            stage_c(g, g % NB)
            if g + 1 < nch:
                stage_b(g + 1, (g + 1) % NB)
            if g + 2 < nch:
                stage_a(g + 2, (g + 2) % NB, g + 2 >= NB)
        for g in range(nch - NB, nch):
            blk = base_blk + g
            pltpu.make_async_copy(tv.at[g % NB], out_dst(blk),
                                  osem.at[g % NB]).wait()

    return _sc_body


def _sc_gather(xs, table_lin, h_base, n_tok):
    nch = n_tok // (NW * CHUNK)
    mesh = plsc.VectorSubcoreMesh(
        core_axis_name="c", subcore_axis_name="s", num_cores=NC, num_subcores=NS
    )
    f = pl.kernel(
        _make_sc_body(h_base, nch),
        out_type=jax.ShapeDtypeStruct((n_tok, 2 * DIM), jnp.float32),
        mesh=mesh,
        scratch_types=[
            pltpu.VMEM((NB, CHUNK), jnp.int32),
            pltpu.VMEM((NB, CHUNK, DIM), jnp.float32),
            pltpu.SemaphoreType.DMA((NB,)),
            pltpu.SemaphoreType.DMA((NB,)),
            pltpu.SemaphoreType.DMA((NB,)),
        ],
        compiler_params=pltpu.CompilerParams(use_tc_tiling_on_sc=False),
    )
    return f(xs, table_lin)


VOCAB = 100000
VB = 2048
NVB = (VOCAB + VB - 1) // VB
TROWS = NVB * (VB // 2)


def _tt_body(tt_ref, o_ref):
    tr = jnp.transpose(tt_ref[...], (1, 0))
    o_ref[:, 0:DIM] = tr[0:VB // 2]
    o_ref[:, DIM:2 * DIM] = tr[VB // 2:VB]


def _table_transpose(table_t):
    return pl.pallas_call(
        _tt_body,
        grid=(NVB,),
        in_specs=[pl.BlockSpec((DIM, VB), lambda i: (0, i))],
        out_specs=pl.BlockSpec((VB // 2, 2 * DIM), lambda i: (i, 0)),
        out_shape=jax.ShapeDtypeStruct((TROWS, 2 * DIM), jnp.float32),
    )(table_t)


SB = 8


def _tc_body(tok_ref, maskt_ref, pos_ref, seg_ref, o_ref):
    tok = tok_ref[:, :, 0:DIM]
    tt = jnp.transpose(tok, (0, 2, 1))
    m = maskt_ref[...].astype(jnp.float32)[:, None, :]
    p3 = pos_ref[...][:, :, None]
    s0 = seg_ref[0:1, :]
    d = seg_ref[1:2, :] - s0
    s03 = s0[:, :, None]
    d3 = d[:, :, None]
    o_ref[...] = tt + p3 + s03 + m * d3


def _tc_body_aliased(tok_ref, maskt_ref, pos_ref, seg_ref, prev_ref, o_ref):
    _tc_body(tok_ref, maskt_ref, pos_ref, seg_ref, o_ref)


def _tc_finish(tok3, mask_t, pos, seg_table, s_off, prev=None):
    nsb = tok3.shape[0] // SB
    off = s_off // SB
    specs = [
        pl.BlockSpec((SB, BATCH, 2 * DIM), lambda i: (i, 0, 0)),
        pl.BlockSpec((SB, BATCH), lambda i: (i, 0)),
        pl.BlockSpec((SB, DIM), lambda i: (i, 0)),
        pl.BlockSpec((2, DIM), lambda i: (0, 0)),
    ]
    args = [tok3, mask_t, pos, seg_table]
    body = _tc_body
    aliases = {}
    if prev is not None:
        specs.append(pl.BlockSpec(memory_space=pl.ANY))
        args.append(prev)
        body = _tc_body_aliased
        aliases = {4: 0}
    return pl.pallas_call(
        body,
        grid=(nsb,),
        in_specs=specs,
        out_specs=pl.BlockSpec((SB, DIM, BATCH), lambda i: (i + off, 0, 0)),
        out_shape=jax.ShapeDtypeStruct((SEQ, DIM, BATCH), jnp.float32),
        input_output_aliases=aliases,
    )(*args)


SPLITS = (48, 48, 48, 56)


def kernel(x, segment_mask, token_table, pos_table, seg_table):
    x = x.astype(jnp.int32)
    segment_mask = segment_mask.astype(jnp.int32)
    xs = x.T.reshape(N // CHUNK, CHUNK)
    mask_t = segment_mask.T
    pos = pos_table[:SEQ]

    table_lin = _table_transpose(token_table.T).reshape(2 * TROWS, DIM)

    toks = []
    s0 = 0
    for ns in SPLITS:
        n_tok = ns * BATCH
        tok = _sc_gather(xs, table_lin, s0 * BATCH // CHUNK, n_tok)
        toks.append(tok.reshape(ns, BATCH, 2 * DIM))
        s0 += ns

    out = None
    s0 = 0
    for ns, tok3 in zip(SPLITS, toks):
        out = _tc_finish(tok3, mask_t[s0:s0 + ns], pos[s0:s0 + ns], seg_table,
                         s0, out)
        s0 += ns
    return out.transpose(2, 0, 1)
